# Initial kernel scaffold; baseline (speedup 1.0000x reference)
import functools

import jax
import jax.numpy as jnp
from jax import lax
from jax.experimental import pallas as pl
from jax.experimental.pallas import tpu as pltpu
from jax.experimental.pallas import tpu_sc as plsc

NN = 10000
EE = 320000
DD = 128

NC = 2
NS = 16
NW = NC * NS
LANES = 16

EPW = EE // NW
CHUNK = 400
ROWS_A = 624
ROWS_B = NN - 15 * ROWS_A


def _matmul_body(v_ref, w_ref, o_ref):
    o_ref[...] = jax.lax.dot_general(
        v_ref[...], w_ref[...],
        dimension_numbers=(((1,), (1,)), ((), ())),
        preferred_element_type=jnp.float32,
    )


def _combine_body(p0_ref, p1_ref, o_ref):
    o_ref[...] = jnp.maximum(p0_ref[...] + p1_ref[...], 0.0)


def _sc_body(support_hbm, src_hbm, dst_hbm, w_hbm, zeros_hbm, out_hbm,
             acc, src_v, dst_v, w_v, rows_v, sem):
    c = lax.axis_index("c")
    s = lax.axis_index("s")
    wid = s * NC + c

    @pl.when(s < NS - 1)
    def _():
        pltpu.sync_copy(zeros_hbm.at[pl.ds(0, ROWS_A)],
                        acc.at[pl.ds(s * ROWS_A, ROWS_A)])

    @pl.when(s == NS - 1)
    def _():
        pltpu.sync_copy(zeros_hbm,
                        acc.at[pl.ds((NS - 1) * ROWS_A, ROWS_B)])

    plsc.subcore_barrier()

    base0 = wid * EPW

    @pl.loop(0, EPW, step=CHUNK)
    def _(i):
        base = base0 + i
        pltpu.sync_copy(src_hbm.at[pl.ds(base, CHUNK)], src_v)
        pltpu.sync_copy(dst_hbm.at[pl.ds(base, CHUNK)], dst_v)
        pltpu.sync_copy(w_hbm.at[pl.ds(base, CHUNK)], w_v)
        pltpu.async_copy(support_hbm.at[src_v], rows_v, sem).wait()

        @pl.loop(0, CHUNK)
        def _(e):
            bw = plsc.load_gather(w_v, [jnp.broadcast_to(e, (LANES,))])
            for k in range(DD // LANES):
                sl = pl.ds(k * LANES, LANES)
                rows_v[e, sl] = rows_v[e, sl] * bw

        pltpu.sync_copy(rows_v, acc.at[dst_v], add=True)

    plsc.subcore_barrier()

    @pl.when(s < NS - 1)
    def _():
        pltpu.sync_copy(acc.at[pl.ds(s * ROWS_A, ROWS_A)],
                        out_hbm.at[pl.ds(c * NN + s * ROWS_A, ROWS_A)])

    @pl.when(s == NS - 1)
    def _():
        pltpu.sync_copy(acc.at[pl.ds((NS - 1) * ROWS_A, ROWS_B)],
                        out_hbm.at[pl.ds(c * NN + (NS - 1) * ROWS_A, ROWS_B)])


@functools.partial(
    pl.kernel,
    out_type=jax.ShapeDtypeStruct((2 * NN, DD), jnp.float32),
    mesh=plsc.VectorSubcoreMesh(core_axis_name="c", subcore_axis_name="s"),
    scratch_types=[
        pltpu.VMEM_SHARED((NN, DD), jnp.float32),
        pltpu.VMEM((CHUNK,), jnp.int32),
        pltpu.VMEM((CHUNK,), jnp.int32),
        pltpu.VMEM((CHUNK,), jnp.float32),
        pltpu.VMEM((CHUNK, DD), jnp.float32),
        pltpu.SemaphoreType.DMA,
    ],
)
def _sc_spmm(support_hbm, src_hbm, dst_hbm, w_hbm, zeros_hbm, out_hbm,
             acc, src_v, dst_v, w_v, rows_v, sem):
    _sc_body(support_hbm, src_hbm, dst_hbm, w_hbm, zeros_hbm, out_hbm,
             acc, src_v, dst_v, w_v, rows_v, sem)


def kernel(v, edge_index, edge_weight, W):
    support = pl.pallas_call(
        _matmul_body,
        grid=(10,),
        in_specs=[
            pl.BlockSpec((NN // 10, DD), lambda i: (i, 0)),
            pl.BlockSpec((DD, DD), lambda i: (0, 0)),
        ],
        out_specs=pl.BlockSpec((NN // 10, DD), lambda i: (i, 0)),
        out_shape=jax.ShapeDtypeStruct((NN, DD), jnp.float32),
    )(v, W)

    src = edge_index[0]
    dst = edge_index[1]
    zeros = jnp.zeros((ROWS_B, DD), jnp.float32)

    partial = _sc_spmm(support, src, dst, edge_weight, zeros)

    out = pl.pallas_call(
        _combine_body,
        grid=(10,),
        in_specs=[
            pl.BlockSpec((NN // 10, DD), lambda i: (i, 0)),
            pl.BlockSpec((NN // 10, DD), lambda i: (i, 0)),
        ],
        out_specs=pl.BlockSpec((NN // 10, DD), lambda i: (i, 0)),
        out_shape=jax.ShapeDtypeStruct((NN, DD), jnp.float32),
    )(partial[:NN], partial[NN:])
    return out


# speedup vs baseline: 4.9506x; 4.9506x over previous
import functools

import jax
import jax.numpy as jnp
from jax import lax
from jax.experimental import pallas as pl
from jax.experimental.pallas import tpu as pltpu
from jax.experimental.pallas import tpu_sc as plsc

NN = 10000
EE = 320000
DD = 128

NC = 2
NS = 16
NW = NC * NS
LANES = 16

EPW = EE // NW
CHUNK = 200
ROWS_A = 624
ROWS_B = NN - 15 * ROWS_A


def _matmul_body(v_ref, w_ref, o_ref):
    o_ref[...] = jax.lax.dot_general(
        v_ref[...], w_ref[...],
        dimension_numbers=(((1,), (1,)), ((), ())),
        preferred_element_type=jnp.float32,
    )


def _combine_body(p0_ref, p1_ref, o_ref):
    o_ref[...] = jnp.maximum(p0_ref[...] + p1_ref[...], 0.0)


def _sc_body(support_hbm, src_hbm, dst_hbm, w_hbm, zeros_hbm, out_hbm,
             acc, src_v, dst_v, w_v, rows_v, sem):
    c = lax.axis_index("c")
    s = lax.axis_index("s")
    wid = s * NC + c

    @pl.when(s < NS - 1)
    def _():
        pltpu.sync_copy(zeros_hbm.at[pl.ds(0, ROWS_A)],
                        acc.at[pl.ds(s * ROWS_A, ROWS_A)])

    @pl.when(s == NS - 1)
    def _():
        pltpu.sync_copy(zeros_hbm,
                        acc.at[pl.ds((NS - 1) * ROWS_A, ROWS_B)])

    plsc.subcore_barrier()

    base0 = wid * EPW

    @pl.loop(0, EPW, step=CHUNK)
    def _(i):
        base = base0 + i
        pltpu.sync_copy(src_hbm.at[pl.ds(base, CHUNK)], src_v)
        pltpu.sync_copy(dst_hbm.at[pl.ds(base, CHUNK)], dst_v)
        pltpu.sync_copy(w_hbm.at[pl.ds(base, CHUNK)], w_v)
        pltpu.async_copy(support_hbm.at[src_v], rows_v, sem).wait()

        @pl.loop(0, CHUNK)
        def _(e):
            bw = plsc.load_gather(w_v, [jnp.broadcast_to(e, (LANES,))])
            for k in range(DD // LANES):
                sl = pl.ds(k * LANES, LANES)
                rows_v[e, sl] = rows_v[e, sl] * bw

        pltpu.sync_copy(rows_v, acc.at[dst_v], add=True)

    plsc.subcore_barrier()

    @pl.when(s < NS - 1)
    def _():
        pltpu.sync_copy(acc.at[pl.ds(s * ROWS_A, ROWS_A)],
                        out_hbm.at[pl.ds(c * NN + s * ROWS_A, ROWS_A)])

    @pl.when(s == NS - 1)
    def _():
        pltpu.sync_copy(acc.at[pl.ds((NS - 1) * ROWS_A, ROWS_B)],
                        out_hbm.at[pl.ds(c * NN + (NS - 1) * ROWS_A, ROWS_B)])


@functools.partial(
    pl.kernel,
    out_type=jax.ShapeDtypeStruct((2 * NN, DD), jnp.float32),
    mesh=plsc.VectorSubcoreMesh(core_axis_name="c", subcore_axis_name="s"),
    scratch_types=[
        pltpu.VMEM_SHARED((NN, DD), jnp.float32),
        pltpu.VMEM((CHUNK,), jnp.int32),
        pltpu.VMEM((CHUNK,), jnp.int32),
        pltpu.VMEM((CHUNK,), jnp.float32),
        pltpu.VMEM((CHUNK, DD), jnp.float32),
        pltpu.SemaphoreType.DMA,
    ],
    compiler_params=pltpu.CompilerParams(needs_layout_passes=False),
)
def _sc_spmm(support_hbm, src_hbm, dst_hbm, w_hbm, zeros_hbm, out_hbm,
             acc, src_v, dst_v, w_v, rows_v, sem):
    _sc_body(support_hbm, src_hbm, dst_hbm, w_hbm, zeros_hbm, out_hbm,
             acc, src_v, dst_v, w_v, rows_v, sem)


def kernel(v, edge_index, edge_weight, W):
    support = pl.pallas_call(
        _matmul_body,
        grid=(10,),
        in_specs=[
            pl.BlockSpec((NN // 10, DD), lambda i: (i, 0)),
            pl.BlockSpec((DD, DD), lambda i: (0, 0)),
        ],
        out_specs=pl.BlockSpec((NN // 10, DD), lambda i: (i, 0)),
        out_shape=jax.ShapeDtypeStruct((NN, DD), jnp.float32),
    )(v, W)

    src = edge_index[0]
    dst = edge_index[1]
    zeros = jnp.zeros((ROWS_B, DD), jnp.float32)

    partial = _sc_spmm(support, src, dst, edge_weight, zeros)

    out = pl.pallas_call(
        _combine_body,
        grid=(10,),
        in_specs=[
            pl.BlockSpec((NN // 10, DD), lambda i: (i, 0)),
            pl.BlockSpec((NN // 10, DD), lambda i: (i, 0)),
        ],
        out_specs=pl.BlockSpec((NN // 10, DD), lambda i: (i, 0)),
        out_shape=jax.ShapeDtypeStruct((NN, DD), jnp.float32),
    )(partial[:NN], partial[NN:])
    return out


# speedup vs baseline: 7.5563x; 1.5263x over previous
import functools

import jax
import jax.numpy as jnp
from jax import lax
from jax.experimental import pallas as pl
from jax.experimental.pallas import tpu as pltpu
from jax.experimental.pallas import tpu_sc as plsc

NN = 10000
EE = 320000
DD = 128

NC = 2
NS = 16
NW = NC * NS
LANES = 16

EPW = EE // NW
NPHASE = 2
EPP = EPW // NPHASE
CHUNK = 40
NCHUNK = EPP // CHUNK
ROWS_A = 624
ROWS_B = NN - 15 * ROWS_A


def _matmul_body(v_ref, w_ref, o_ref):
    o_ref[...] = jax.lax.dot_general(
        v_ref[...], w_ref[...],
        dimension_numbers=(((1,), (1,)), ((), ())),
        preferred_element_type=jnp.float32,
    )


def _combine_body(p0_ref, p1_ref, o_ref):
    o_ref[...] = jnp.maximum(p0_ref[...] + p1_ref[...], 0.0)


def _sc_body(support_hbm, src_hbm, dst_hbm, w_hbm, zeros_hbm, out_hbm,
             acc, src_v, dst_v, w_v, rows, gsem, ssem):
    c = lax.axis_index("c")
    s = lax.axis_index("s")
    wid = s * NC + c

    @pl.when(s < NS - 1)
    def _():
        pltpu.sync_copy(zeros_hbm.at[pl.ds(0, ROWS_A)],
                        acc.at[pl.ds(s * ROWS_A, ROWS_A)])

    @pl.when(s == NS - 1)
    def _():
        pltpu.sync_copy(zeros_hbm,
                        acc.at[pl.ds((NS - 1) * ROWS_A, ROWS_B)])

    plsc.subcore_barrier()

    def issue_gather(j, b):
        pltpu.async_copy(
            support_hbm.at[src_v.at[pl.ds(j * CHUNK, CHUNK)]],
            rows[b], gsem.at[b])

    def wait_gather(b):
        pltpu.make_async_copy(
            support_hbm.at[src_v.at[pl.ds(0, CHUNK)]],
            rows[b], gsem.at[b]).wait()

    def wait_scatter(b):
        pltpu.make_async_copy(rows[b], acc.at[dst_v.at[0]], ssem.at[b]).wait()

    def sub(j, a, nxt, guard, gather):
        wait_gather(a)
        if guard:
            wait_scatter(nxt)
        if gather:
            issue_gather(j + 2, nxt)
        ebase = j * CHUNK

        @pl.loop(0, CHUNK)
        def _(e):
            bw = plsc.load_gather(
                w_v, [jnp.broadcast_to(ebase + e, (LANES,))])
            for k in range(DD // LANES):
                sl = pl.ds(k * LANES, LANES)
                rows[a][e, sl] = rows[a][e, sl] * bw

        pltpu.async_copy(rows[a], acc.at[dst_v.at[j]], ssem.at[a], add=True)

    for p in range(NPHASE):
        base = wid * EPW + p * EPP
        pltpu.sync_copy(src_hbm.at[pl.ds(base, EPP)], src_v)
        pltpu.sync_copy(w_hbm.at[pl.ds(base, EPP)], w_v)
        pltpu.sync_copy(dst_hbm.at[wid * NPHASE + p], dst_v)

        issue_gather(0, 0)
        issue_gather(1, 1)
        sub(0, 0, 2, guard=False, gather=True)
        sub(1, 1, 0, guard=True, gather=True)

        @pl.loop(2, NCHUNK - 5, step=3)
        def _(j):
            sub(j, 2, 1, guard=True, gather=True)
            sub(j + 1, 0, 2, guard=True, gather=True)
            sub(j + 2, 1, 0, guard=True, gather=True)

        sub(NCHUNK - 3, 2, 1, guard=True, gather=True)
        sub(NCHUNK - 2, 0, 2, guard=True, gather=False)
        sub(NCHUNK - 1, 1, 0, guard=True, gather=False)

        wait_scatter(1)

    plsc.subcore_barrier()

    @pl.when(s < NS - 1)
    def _():
        pltpu.sync_copy(acc.at[pl.ds(s * ROWS_A, ROWS_A)],
                        out_hbm.at[pl.ds(c * NN + s * ROWS_A, ROWS_A)])

    @pl.when(s == NS - 1)
    def _():
        pltpu.sync_copy(acc.at[pl.ds((NS - 1) * ROWS_A, ROWS_B)],
                        out_hbm.at[pl.ds(c * NN + (NS - 1) * ROWS_A, ROWS_B)])


@functools.partial(
    pl.kernel,
    out_type=jax.ShapeDtypeStruct((2 * NN, DD), jnp.float32),
    mesh=plsc.VectorSubcoreMesh(core_axis_name="c", subcore_axis_name="s"),
    scratch_types=[
        pltpu.VMEM_SHARED((NN, DD), jnp.float32),
        pltpu.VMEM((EPP,), jnp.int32),
        pltpu.VMEM((NCHUNK, CHUNK), jnp.int32),
        pltpu.VMEM((EPP,), jnp.float32),
        pltpu.VMEM((CHUNK, DD), jnp.float32),
        pltpu.VMEM((CHUNK, DD), jnp.float32),
        pltpu.VMEM((CHUNK, DD), jnp.float32),
        pltpu.SemaphoreType.DMA((3,)),
        pltpu.SemaphoreType.DMA((3,)),
    ],
    compiler_params=pltpu.CompilerParams(needs_layout_passes=False),
)
def _sc_spmm(support_hbm, src_hbm, dst_hbm, w_hbm, zeros_hbm, out_hbm,
             acc, src_v, dst_v, w_v, rows0, rows1, rows2, gsem, ssem):
    _sc_body(support_hbm, src_hbm, dst_hbm, w_hbm, zeros_hbm, out_hbm,
             acc, src_v, dst_v, w_v, (rows0, rows1, rows2), gsem, ssem)


def kernel(v, edge_index, edge_weight, W):
    support = pl.pallas_call(
        _matmul_body,
        grid=(10,),
        in_specs=[
            pl.BlockSpec((NN // 10, DD), lambda i: (i, 0)),
            pl.BlockSpec((DD, DD), lambda i: (0, 0)),
        ],
        out_specs=pl.BlockSpec((NN // 10, DD), lambda i: (i, 0)),
        out_shape=jax.ShapeDtypeStruct((NN, DD), jnp.float32),
    )(v, W)

    src = edge_index[0]
    dst = edge_index[1].reshape(NW * NPHASE, NCHUNK, CHUNK)
    zeros = jnp.zeros((ROWS_B, DD), jnp.float32)

    partial = _sc_spmm(support, src, dst, edge_weight, zeros)

    out = pl.pallas_call(
        _combine_body,
        grid=(10,),
        in_specs=[
            pl.BlockSpec((NN // 10, DD), lambda i: (i, 0)),
            pl.BlockSpec((NN // 10, DD), lambda i: (i, 0)),
        ],
        out_specs=pl.BlockSpec((NN // 10, DD), lambda i: (i, 0)),
        out_shape=jax.ShapeDtypeStruct((NN, DD), jnp.float32),
    )(partial[:NN], partial[NN:])
    return out


# speedup vs baseline: 9.2297x; 1.2215x over previous
import functools

import jax
import jax.numpy as jnp
from jax import lax
from jax.experimental import pallas as pl
from jax.experimental.pallas import tpu as pltpu
from jax.experimental.pallas import tpu_sc as plsc

NN = 10000
EE = 320000
DD = 128

NC = 2
NS = 16
NW = NC * NS
LANES = 16

EPW = EE // NW
NPHASE = 2
EPP = EPW // NPHASE
CHUNK = 40
NCHUNK = EPP // CHUNK
ROWS_A = 624
ROWS_B = NN - 15 * ROWS_A


def _matmul_body(v_ref, w_ref, o_ref):
    o_ref[...] = jax.lax.dot_general(
        v_ref[...], w_ref[...],
        dimension_numbers=(((1,), (1,)), ((), ())),
        preferred_element_type=jnp.float32,
    )


def _combine_body(p0_ref, p1_ref, o_ref):
    o_ref[...] = jnp.maximum(p0_ref[...] + p1_ref[...], 0.0)


def _sc_body(support_hbm, src_hbm, dst_hbm, w_hbm, zeros_hbm, out_hbm,
             acc, src_v, dst_v, w_v, rows, gsem, ssem):
    c = lax.axis_index("c")
    s = lax.axis_index("s")
    wid = s * NC + c

    @pl.when(s < NS - 1)
    def _():
        pltpu.sync_copy(zeros_hbm.at[pl.ds(0, ROWS_A)],
                        acc.at[pl.ds(s * ROWS_A, ROWS_A)])

    @pl.when(s == NS - 1)
    def _():
        pltpu.sync_copy(zeros_hbm,
                        acc.at[pl.ds((NS - 1) * ROWS_A, ROWS_B)])

    plsc.subcore_barrier()

    def issue_gather(j, b):
        pltpu.async_copy(
            support_hbm.at[src_v.at[pl.ds(j * CHUNK, CHUNK)]],
            rows[b], gsem.at[b])

    def wait_gather(b):
        pltpu.make_async_copy(
            support_hbm.at[src_v.at[pl.ds(0, CHUNK)]],
            rows[b], gsem.at[b]).wait()

    def wait_scatter(b):
        pltpu.make_async_copy(rows[b], acc.at[dst_v.at[0]], ssem.at[b]).wait()

    def sub(j, a, nxt, guard, gather):
        wait_gather(a)
        if guard:
            wait_scatter(nxt)
        if gather:
            issue_gather(j + 2, nxt)
        ebase = j * CHUNK

        @plsc.parallel_loop(0, CHUNK, unroll=4)
        def _(e):
            bw = plsc.load_gather(
                w_v, [jnp.broadcast_to(ebase + e, (LANES,))])
            for k in range(DD // LANES):
                sl = pl.ds(k * LANES, LANES)
                rows[a][e, sl] = rows[a][e, sl] * bw

        pltpu.async_copy(rows[a], acc.at[dst_v.at[j]], ssem.at[a], add=True)

    for p in range(NPHASE):
        base = wid * EPW + p * EPP
        pltpu.sync_copy(src_hbm.at[pl.ds(base, EPP)], src_v)
        pltpu.sync_copy(w_hbm.at[pl.ds(base, EPP)], w_v)
        pltpu.sync_copy(dst_hbm.at[wid * NPHASE + p], dst_v)

        issue_gather(0, 0)
        issue_gather(1, 1)
        sub(0, 0, 2, guard=False, gather=True)
        sub(1, 1, 0, guard=True, gather=True)

        @pl.loop(2, NCHUNK - 5, step=3)
        def _(j):
            sub(j, 2, 1, guard=True, gather=True)
            sub(j + 1, 0, 2, guard=True, gather=True)
            sub(j + 2, 1, 0, guard=True, gather=True)

        sub(NCHUNK - 3, 2, 1, guard=True, gather=True)
        sub(NCHUNK - 2, 0, 2, guard=True, gather=False)
        sub(NCHUNK - 1, 1, 0, guard=True, gather=False)

        wait_scatter(1)

    plsc.subcore_barrier()

    @pl.when(s < NS - 1)
    def _():
        pltpu.sync_copy(acc.at[pl.ds(s * ROWS_A, ROWS_A)],
                        out_hbm.at[pl.ds(c * NN + s * ROWS_A, ROWS_A)])

    @pl.when(s == NS - 1)
    def _():
        pltpu.sync_copy(acc.at[pl.ds((NS - 1) * ROWS_A, ROWS_B)],
                        out_hbm.at[pl.ds(c * NN + (NS - 1) * ROWS_A, ROWS_B)])


@functools.partial(
    pl.kernel,
    out_type=jax.ShapeDtypeStruct((2 * NN, DD), jnp.float32),
    mesh=plsc.VectorSubcoreMesh(core_axis_name="c", subcore_axis_name="s"),
    scratch_types=[
        pltpu.VMEM_SHARED((NN, DD), jnp.float32),
        pltpu.VMEM((EPP,), jnp.int32),
        pltpu.VMEM((NCHUNK, CHUNK), jnp.int32),
        pltpu.VMEM((EPP,), jnp.float32),
        pltpu.VMEM((CHUNK, DD), jnp.float32),
        pltpu.VMEM((CHUNK, DD), jnp.float32),
        pltpu.VMEM((CHUNK, DD), jnp.float32),
        pltpu.SemaphoreType.DMA((3,)),
        pltpu.SemaphoreType.DMA((3,)),
    ],
    compiler_params=pltpu.CompilerParams(needs_layout_passes=False),
)
def _sc_spmm(support_hbm, src_hbm, dst_hbm, w_hbm, zeros_hbm, out_hbm,
             acc, src_v, dst_v, w_v, rows0, rows1, rows2, gsem, ssem):
    _sc_body(support_hbm, src_hbm, dst_hbm, w_hbm, zeros_hbm, out_hbm,
             acc, src_v, dst_v, w_v, (rows0, rows1, rows2), gsem, ssem)


def kernel(v, edge_index, edge_weight, W):
    support = pl.pallas_call(
        _matmul_body,
        grid=(10,),
        in_specs=[
            pl.BlockSpec((NN // 10, DD), lambda i: (i, 0)),
            pl.BlockSpec((DD, DD), lambda i: (0, 0)),
        ],
        out_specs=pl.BlockSpec((NN // 10, DD), lambda i: (i, 0)),
        out_shape=jax.ShapeDtypeStruct((NN, DD), jnp.float32),
    )(v, W)

    src = edge_index[0]
    dst = edge_index[1].reshape(NW * NPHASE, NCHUNK, CHUNK)
    zeros = jnp.zeros((ROWS_B, DD), jnp.float32)

    partial = _sc_spmm(support, src, dst, edge_weight, zeros)

    out = pl.pallas_call(
        _combine_body,
        grid=(10,),
        in_specs=[
            pl.BlockSpec((NN // 10, DD), lambda i: (i, 0)),
            pl.BlockSpec((NN // 10, DD), lambda i: (i, 0)),
        ],
        out_specs=pl.BlockSpec((NN // 10, DD), lambda i: (i, 0)),
        out_shape=jax.ShapeDtypeStruct((NN, DD), jnp.float32),
    )(partial[:NN], partial[NN:])
    return out


# speedup vs baseline: 9.7304x; 1.0542x over previous
import functools

import jax
import jax.numpy as jnp
from jax import lax
from jax.experimental import pallas as pl
from jax.experimental.pallas import tpu as pltpu
from jax.experimental.pallas import tpu_sc as plsc

NN = 10000
EE = 320000
DD = 128

NC = 2
NS = 16
NW = NC * NS
LANES = 16

EPW = EE // NW
NPHASE = 5
EPP = EPW // NPHASE
CHUNK = 80
NCHUNK = EPP // CHUNK
ROWS_A = 624
ROWS_B = NN - 15 * ROWS_A


def _matmul_body(v_ref, w_ref, o_ref):
    o_ref[...] = jax.lax.dot_general(
        v_ref[...], w_ref[...],
        dimension_numbers=(((1,), (1,)), ((), ())),
        preferred_element_type=jnp.float32,
    )


def _combine_body(p0_ref, p1_ref, o_ref):
    o_ref[...] = jnp.maximum(p0_ref[...] + p1_ref[...], 0.0)


def _sc_body(support_hbm, src_hbm, dst_hbm, w_hbm, zeros_hbm, out_hbm,
             acc, src_v, dst_v, w_v, rows, gsem, ssem):
    c = lax.axis_index("c")
    s = lax.axis_index("s")
    wid = s * NC + c

    @pl.when(s < NS - 1)
    def _():
        pltpu.sync_copy(zeros_hbm.at[pl.ds(0, ROWS_A)],
                        acc.at[pl.ds(s * ROWS_A, ROWS_A)])

    @pl.when(s == NS - 1)
    def _():
        pltpu.sync_copy(zeros_hbm,
                        acc.at[pl.ds((NS - 1) * ROWS_A, ROWS_B)])

    plsc.subcore_barrier()

    def issue_gather(j, b):
        pltpu.async_copy(
            support_hbm.at[src_v.at[pl.ds(j * CHUNK, CHUNK)]],
            rows[b], gsem.at[b])

    def wait_gather(b):
        pltpu.make_async_copy(
            support_hbm.at[src_v.at[pl.ds(0, CHUNK)]],
            rows[b], gsem.at[b]).wait()

    def wait_scatter(b):
        pltpu.make_async_copy(rows[b], acc.at[dst_v.at[0]], ssem.at[b]).wait()

    def sub(j, a, nxt, guard, gather):
        wait_gather(a)
        if guard:
            wait_scatter(nxt)
        if gather:
            issue_gather(j + 2, nxt)
        ebase = j * CHUNK

        @plsc.parallel_loop(0, CHUNK, unroll=4)
        def _(e):
            bw = plsc.load_gather(
                w_v, [jnp.broadcast_to(ebase + e, (LANES,))])
            for k in range(DD // LANES):
                sl = pl.ds(k * LANES, LANES)
                rows[a][e, sl] = rows[a][e, sl] * bw

        pltpu.async_copy(rows[a], acc.at[dst_v.at[j]], ssem.at[a], add=True)

    for p in range(NPHASE):
        base = wid * EPW + p * EPP
        pltpu.sync_copy(src_hbm.at[pl.ds(base, EPP)], src_v)
        pltpu.sync_copy(w_hbm.at[pl.ds(base, EPP)], w_v)
        pltpu.sync_copy(dst_hbm.at[wid * NPHASE + p], dst_v)

        issue_gather(0, 0)
        issue_gather(1, 1)
        sub(0, 0, 2, guard=False, gather=True)
        sub(1, 1, 0, guard=True, gather=True)

        @pl.loop(2, NCHUNK - 5, step=3)
        def _(j):
            sub(j, 2, 1, guard=True, gather=True)
            sub(j + 1, 0, 2, guard=True, gather=True)
            sub(j + 2, 1, 0, guard=True, gather=True)

        sub(NCHUNK - 3, 2, 1, guard=True, gather=True)
        sub(NCHUNK - 2, 0, 2, guard=True, gather=False)
        sub(NCHUNK - 1, 1, 0, guard=True, gather=False)

        wait_scatter(1)

    plsc.subcore_barrier()

    @pl.when(s < NS - 1)
    def _():
        pltpu.sync_copy(acc.at[pl.ds(s * ROWS_A, ROWS_A)],
                        out_hbm.at[pl.ds(c * NN + s * ROWS_A, ROWS_A)])

    @pl.when(s == NS - 1)
    def _():
        pltpu.sync_copy(acc.at[pl.ds((NS - 1) * ROWS_A, ROWS_B)],
                        out_hbm.at[pl.ds(c * NN + (NS - 1) * ROWS_A, ROWS_B)])


@functools.partial(
    pl.kernel,
    out_type=jax.ShapeDtypeStruct((2 * NN, DD), jnp.float32),
    mesh=plsc.VectorSubcoreMesh(core_axis_name="c", subcore_axis_name="s"),
    scratch_types=[
        pltpu.VMEM_SHARED((NN, DD), jnp.float32),
        pltpu.VMEM((EPP,), jnp.int32),
        pltpu.VMEM((NCHUNK, CHUNK), jnp.int32),
        pltpu.VMEM((EPP,), jnp.float32),
        pltpu.VMEM((CHUNK, DD), jnp.float32),
        pltpu.VMEM((CHUNK, DD), jnp.float32),
        pltpu.VMEM((CHUNK, DD), jnp.float32),
        pltpu.SemaphoreType.DMA((3,)),
        pltpu.SemaphoreType.DMA((3,)),
    ],
    compiler_params=pltpu.CompilerParams(needs_layout_passes=False),
)
def _sc_spmm(support_hbm, src_hbm, dst_hbm, w_hbm, zeros_hbm, out_hbm,
             acc, src_v, dst_v, w_v, rows0, rows1, rows2, gsem, ssem):
    _sc_body(support_hbm, src_hbm, dst_hbm, w_hbm, zeros_hbm, out_hbm,
             acc, src_v, dst_v, w_v, (rows0, rows1, rows2), gsem, ssem)


def kernel(v, edge_index, edge_weight, W):
    support = pl.pallas_call(
        _matmul_body,
        grid=(10,),
        in_specs=[
            pl.BlockSpec((NN // 10, DD), lambda i: (i, 0)),
            pl.BlockSpec((DD, DD), lambda i: (0, 0)),
        ],
        out_specs=pl.BlockSpec((NN // 10, DD), lambda i: (i, 0)),
        out_shape=jax.ShapeDtypeStruct((NN, DD), jnp.float32),
    )(v, W)

    src = edge_index[0]
    dst = edge_index[1].reshape(NW * NPHASE, NCHUNK, CHUNK)
    zeros = jnp.zeros((ROWS_B, DD), jnp.float32)

    partial = _sc_spmm(support, src, dst, edge_weight, zeros)

    out = pl.pallas_call(
        _combine_body,
        grid=(10,),
        in_specs=[
            pl.BlockSpec((NN // 10, DD), lambda i: (i, 0)),
            pl.BlockSpec((NN // 10, DD), lambda i: (i, 0)),
        ],
        out_specs=pl.BlockSpec((NN // 10, DD), lambda i: (i, 0)),
        out_shape=jax.ShapeDtypeStruct((NN, DD), jnp.float32),
    )(partial[:NN], partial[NN:])
    return out
